# Initial kernel scaffold; baseline (speedup 1.0000x reference)
import jax
import jax.numpy as jnp
from jax.experimental import pallas as pl


def kernel(bbox_regression, cls_logits, anchors):
    raise NotImplementedError("write your pallas kernel here")


# speedup vs baseline: 1.2355x; 1.2355x over previous
import math

import jax
import jax.numpy as jnp
from jax.experimental import pallas as pl
from jax.experimental.pallas import tpu as pltpu

_B = 4
_A = 8732
_C = 91
_TOPK = 400
_DETS = 200
_SCORE_THRESH = 0.01
_NMS_THRESH = 0.45
_XFORM_CLIP = math.log(1000.0 / 16.0)
_WX, _WY, _WW, _WH = 10.0, 10.0, 5.0, 5.0
_IMG_H, _IMG_W = 300.0, 300.0
_NEG_INF = float("-inf")


def _prep_body(reg_ref, logit_ref, anc_ref, score_ref, box_ref):
    logits = logit_ref[0]
    m = jnp.max(logits, axis=-1, keepdims=True)
    unnorm = jnp.exp(logits - m)
    scores = unnorm / jnp.sum(unnorm, axis=-1, keepdims=True)
    masked = jnp.where(scores > _SCORE_THRESH, scores, _NEG_INF)
    score_ref[0] = masked

    anc = anc_ref[0]
    reg = reg_ref[0]
    widths = anc[:, 2:3] - anc[:, 0:1]
    heights = anc[:, 3:4] - anc[:, 1:2]
    ctr_x = anc[:, 0:1] + 0.5 * widths
    ctr_y = anc[:, 1:2] + 0.5 * heights
    dx = reg[:, 0:1] / _WX
    dy = reg[:, 1:2] / _WY
    dw = jnp.minimum(reg[:, 2:3] / _WW, _XFORM_CLIP)
    dh = jnp.minimum(reg[:, 3:4] / _WH, _XFORM_CLIP)
    pred_ctr_x = dx * widths + ctr_x
    pred_ctr_y = dy * heights + ctr_y
    pred_w = jnp.exp(dw) * widths
    pred_h = jnp.exp(dh) * heights
    x1 = jnp.clip(pred_ctr_x - 0.5 * pred_w, 0.0, _IMG_W)
    y1 = jnp.clip(pred_ctr_y - 0.5 * pred_h, 0.0, _IMG_H)
    x2 = jnp.clip(pred_ctr_x + 0.5 * pred_w, 0.0, _IMG_W)
    y2 = jnp.clip(pred_ctr_y + 0.5 * pred_h, 0.0, _IMG_H)
    box_ref[0] = jnp.concatenate([x1, y1, x2, y2], axis=-1)


def _prep(reg, logits, anchors):
    return pl.pallas_call(
        _prep_body,
        grid=(_B,),
        in_specs=[
            pl.BlockSpec((1, _A, 4), lambda b: (b, 0, 0)),
            pl.BlockSpec((1, _A, _C), lambda b: (b, 0, 0)),
            pl.BlockSpec((1, _A, 4), lambda b: (b, 0, 0)),
        ],
        out_specs=[
            pl.BlockSpec((1, _A, _C), lambda b: (b, 0, 0)),
            pl.BlockSpec((1, _A, 4), lambda b: (b, 0, 0)),
        ],
        out_shape=[
            jax.ShapeDtypeStruct((_B, _A, _C), jnp.float32),
            jax.ShapeDtypeStruct((_B, _A, 4), jnp.float32),
        ],
    )(reg, logits, anchors)


def _iou_one(box, boxes):
    area1 = (box[2] - box[0]) * (box[3] - box[1])
    area2 = (boxes[:, 2] - boxes[:, 0]) * (boxes[:, 3] - boxes[:, 1])
    lt = jnp.maximum(box[:2], boxes[:, :2])
    rb = jnp.minimum(box[2:], boxes[:, 2:])
    wh = jnp.maximum(rb - lt, 0.0)
    inter = wh[:, 0] * wh[:, 1]
    return inter / (area1 + area2 - inter + 1e-12)


def _greedy_nms(boxes_off, scores):
    def body(i, carry):
        s, keep = carry
        best = jnp.argmax(s)
        keep = keep.at[i].set(best.astype(jnp.int32))
        ious = _iou_one(boxes_off[best], boxes_off)
        s = jnp.where(ious > _NMS_THRESH, _NEG_INF, s)
        s = s.at[best].set(_NEG_INF)
        return (s, keep)
    init = (scores, jnp.zeros((_DETS,), jnp.int32))
    _, keep = jax.lax.fori_loop(0, _DETS, body, init)
    return keep


def _post_one(masked, boxes):
    cls_scores = masked[:, 1:].T
    top_s, top_i = jax.lax.top_k(cls_scores, _TOPK)
    cand_boxes = boxes[top_i.reshape(-1)]
    cand_scores = top_s.reshape(-1)
    labels = jnp.repeat(jnp.arange(1, _C, dtype=jnp.int32), _TOPK)
    max_coord = jnp.max(cand_boxes)
    offsets = labels.astype(jnp.float32) * (max_coord + 1.0)
    boxes_off = cand_boxes + offsets[:, None]
    keep = _greedy_nms(jax.lax.stop_gradient(boxes_off),
                       jax.lax.stop_gradient(cand_scores))
    return cand_boxes[keep], cand_scores[keep], labels[keep]


def kernel(bbox_regression, cls_logits, anchors):
    masked, boxes = _prep(bbox_regression, cls_logits, anchors)
    return jax.vmap(_post_one)(masked, boxes)


# speedup vs baseline: 3.0028x; 2.4304x over previous
import math

import jax
import jax.numpy as jnp
from jax.experimental import pallas as pl
from jax.experimental.pallas import tpu as pltpu

_B = 4
_A = 8732
_C = 91
_TOPK = 400
_DETS = 200
_SCORE_THRESH = 0.01
_NMS_THRESH = 0.45
_XFORM_CLIP = math.log(1000.0 / 16.0)
_WX, _WY, _WW, _WH = 10.0, 10.0, 5.0, 5.0
_IMG_H, _IMG_W = 300.0, 300.0
_NEG_INF = float("-inf")


def _prep_body(reg_ref, logit_ref, anc_ref, score_ref, box_ref):
    logits = logit_ref[0]
    m = jnp.max(logits, axis=-1, keepdims=True)
    unnorm = jnp.exp(logits - m)
    scores = unnorm / jnp.sum(unnorm, axis=-1, keepdims=True)
    masked = jnp.where(scores > _SCORE_THRESH, scores, _NEG_INF)
    score_ref[0] = masked

    anc = anc_ref[0]
    reg = reg_ref[0]
    widths = anc[:, 2:3] - anc[:, 0:1]
    heights = anc[:, 3:4] - anc[:, 1:2]
    ctr_x = anc[:, 0:1] + 0.5 * widths
    ctr_y = anc[:, 1:2] + 0.5 * heights
    dx = reg[:, 0:1] / _WX
    dy = reg[:, 1:2] / _WY
    dw = jnp.minimum(reg[:, 2:3] / _WW, _XFORM_CLIP)
    dh = jnp.minimum(reg[:, 3:4] / _WH, _XFORM_CLIP)
    pred_ctr_x = dx * widths + ctr_x
    pred_ctr_y = dy * heights + ctr_y
    pred_w = jnp.exp(dw) * widths
    pred_h = jnp.exp(dh) * heights
    x1 = jnp.clip(pred_ctr_x - 0.5 * pred_w, 0.0, _IMG_W)
    y1 = jnp.clip(pred_ctr_y - 0.5 * pred_h, 0.0, _IMG_H)
    x2 = jnp.clip(pred_ctr_x + 0.5 * pred_w, 0.0, _IMG_W)
    y2 = jnp.clip(pred_ctr_y + 0.5 * pred_h, 0.0, _IMG_H)
    box_ref[0] = jnp.concatenate([x1, y1, x2, y2], axis=-1)


def _prep(reg, logits, anchors):
    return pl.pallas_call(
        _prep_body,
        grid=(_B,),
        in_specs=[
            pl.BlockSpec((1, _A, 4), lambda b: (b, 0, 0)),
            pl.BlockSpec((1, _A, _C), lambda b: (b, 0, 0)),
            pl.BlockSpec((1, _A, 4), lambda b: (b, 0, 0)),
        ],
        out_specs=[
            pl.BlockSpec((1, _A, _C), lambda b: (b, 0, 0)),
            pl.BlockSpec((1, _A, 4), lambda b: (b, 0, 0)),
        ],
        out_shape=[
            jax.ShapeDtypeStruct((_B, _A, _C), jnp.float32),
            jax.ShapeDtypeStruct((_B, _A, 4), jnp.float32),
        ],
    )(reg, logits, anchors)


_NCAND = (_C - 1) * _TOPK
_NROWS = 288
_NPAD = _NROWS * 128


def _nms_body(score_ref, x1_ref, y1_ref, x2_ref, y2_ref,
              obox_ref, oscore_ref, olabel_ref,
              s_ref, xo1_ref, yo1_ref, xo2_ref, yo2_ref, area_ref, iota_ref):
    iota = (jax.lax.broadcasted_iota(jnp.int32, (_NROWS, 128), 0) * 128
            + jax.lax.broadcasted_iota(jnp.int32, (_NROWS, 128), 1))
    iota_ref[...] = iota
    s_ref[...] = score_ref[0]
    x1 = x1_ref[0]
    y1 = y1_ref[0]
    x2 = x2_ref[0]
    y2 = y2_ref[0]
    max_coord = jnp.maximum(jnp.maximum(jnp.max(x1), jnp.max(y1)),
                            jnp.maximum(jnp.max(x2), jnp.max(y2)))
    labels = (iota // _TOPK + 1)
    off = labels.astype(jnp.float32) * (max_coord + 1.0)
    xo1 = x1 + off
    yo1 = y1 + off
    xo2 = x2 + off
    yo2 = y2 + off
    xo1_ref[...] = xo1
    yo1_ref[...] = yo1
    xo2_ref[...] = xo2
    yo2_ref[...] = yo2
    area_ref[...] = (xo2 - xo1) * (yo2 - yo1)

    def body(i, _):
        s = s_ref[...]
        m = jnp.max(s)
        best = jnp.min(jnp.where(s == m, iota_ref[...], jnp.int32(2**30)))
        r = best // 128
        c = jnp.remainder(best, 128)
        rs = pl.ds(r, 1)
        eq = jax.lax.broadcasted_iota(jnp.int32, (1, 128), 1) == c

        def fetch(row):
            return jnp.sum(jnp.where(eq, row, 0.0), axis=1, keepdims=True)

        bx1 = fetch(xo1_ref[rs, :])
        by1 = fetch(yo1_ref[rs, :])
        bx2 = fetch(xo2_ref[rs, :])
        by2 = fetch(yo2_ref[rs, :])
        area1 = (bx2 - bx1) * (by2 - by1)
        ltx = jnp.maximum(bx1, xo1_ref[...])
        lty = jnp.maximum(by1, yo1_ref[...])
        rbx = jnp.minimum(bx2, xo2_ref[...])
        rby = jnp.minimum(by2, yo2_ref[...])
        w = jnp.maximum(rbx - ltx, 0.0)
        h = jnp.maximum(rby - lty, 0.0)
        inter = w * h
        iou = inter / (area1 + area_ref[...] - inter + 1e-12)
        s = jnp.where(iou > _NMS_THRESH, _NEG_INF, s)
        s_ref[...] = s
        s_ref[rs, :] = jnp.where(eq, _NEG_INF, s_ref[rs, :])
        ob = jnp.concatenate([fetch(x1_ref[0, rs, :]), fetch(y1_ref[0, rs, :]),
                              fetch(x2_ref[0, rs, :]), fetch(y2_ref[0, rs, :])],
                             axis=-1)
        obox_ref[0, pl.ds(i, 1), :] = ob
        oscore_ref[0, pl.ds(i, 1), :] = fetch(score_ref[0, rs, :])
        olabel_ref[0, pl.ds(i, 1), :] = jnp.reshape(best // _TOPK + 1, (1, 1))
        return 0

    jax.lax.fori_loop(0, _DETS, body, 0)


def _nms(scores_pad, x1, y1, x2, y2):
    return pl.pallas_call(
        _nms_body,
        grid=(_B,),
        in_specs=[pl.BlockSpec((1, _NROWS, 128), lambda b: (b, 0, 0))] * 5,
        out_specs=[
            pl.BlockSpec((1, _DETS, 4), lambda b: (b, 0, 0)),
            pl.BlockSpec((1, _DETS, 1), lambda b: (b, 0, 0)),
            pl.BlockSpec((1, _DETS, 1), lambda b: (b, 0, 0)),
        ],
        out_shape=[
            jax.ShapeDtypeStruct((_B, _DETS, 4), jnp.float32),
            jax.ShapeDtypeStruct((_B, _DETS, 1), jnp.float32),
            jax.ShapeDtypeStruct((_B, _DETS, 1), jnp.int32),
        ],
        scratch_shapes=[pltpu.VMEM((_NROWS, 128), jnp.float32)] * 6
        + [pltpu.VMEM((_NROWS, 128), jnp.int32)],
    )(scores_pad, x1, y1, x2, y2)


def kernel(bbox_regression, cls_logits, anchors):
    masked, boxes = _prep(bbox_regression, cls_logits, anchors)

    def topk_one(m, bx):
        cls_scores = m[:, 1:].T
        top_s, top_i = jax.lax.top_k(cls_scores, _TOPK)
        cand_boxes = bx[top_i.reshape(-1)]
        return top_s.reshape(-1), cand_boxes

    cand_scores, cand_boxes = jax.vmap(topk_one)(masked, boxes)

    scores_pad = jnp.pad(cand_scores, ((0, 0), (0, _NPAD - _NCAND)),
                         constant_values=_NEG_INF).reshape(_B, _NROWS, 128)
    planes = jnp.pad(cand_boxes, ((0, 0), (0, _NPAD - _NCAND), (0, 0)))
    planes = jnp.moveaxis(planes, 2, 1).reshape(_B, 4, _NROWS, 128)
    ob, osc, olb = _nms(scores_pad, planes[:, 0], planes[:, 1],
                        planes[:, 2], planes[:, 3])
    return ob, osc[..., 0], olb[..., 0]


# speedup vs baseline: 19.3492x; 6.4437x over previous
import functools
import math

import jax
import jax.numpy as jnp
from jax.experimental import pallas as pl
from jax.experimental.pallas import tpu as pltpu
from jax.experimental.pallas import tpu_sc as plsc

_B = 4
_A = 8732
_APAD = 8736
_C = 91
_NCLS = _C - 1
_NROWSEL = _B * _NCLS
_TOPK = 400
_DETS = 200
_SCORE_THRESH = 0.01
_NMS_THRESH = 0.45
_XFORM_CLIP = math.log(1000.0 / 16.0)
_WX, _WY, _WW, _WH = 10.0, 10.0, 5.0, 5.0
_IMG_H, _IMG_W = 300.0, 300.0
_NEG_INF = float("-inf")
_ONE_KEY = 0x3F800000

_NCAND = _NCLS * _TOPK
_NROWS = 288
_NPAD = _NROWS * 128

_NW = 32
_CHUNKS = _APAD // 16


def _prep_body(reg_ref, logit_ref, anc_ref, key_ref, box_ref):
    logits = logit_ref[0]
    m = jnp.max(logits, axis=-1, keepdims=True)
    unnorm = jnp.exp(logits - m)
    scores = unnorm / jnp.sum(unnorm, axis=-1, keepdims=True)
    cidx = jax.lax.broadcasted_iota(jnp.int32, (_A, _C), 1)
    keys = jnp.where((scores > _SCORE_THRESH) & (cidx > 0),
                     jax.lax.bitcast_convert_type(scores, jnp.int32),
                     jnp.int32(-1))
    key_ref[0] = keys

    anc = anc_ref[0]
    reg = reg_ref[0]
    widths = anc[:, 2:3] - anc[:, 0:1]
    heights = anc[:, 3:4] - anc[:, 1:2]
    ctr_x = anc[:, 0:1] + 0.5 * widths
    ctr_y = anc[:, 1:2] + 0.5 * heights
    dx = reg[:, 0:1] / _WX
    dy = reg[:, 1:2] / _WY
    dw = jnp.minimum(reg[:, 2:3] / _WW, _XFORM_CLIP)
    dh = jnp.minimum(reg[:, 3:4] / _WH, _XFORM_CLIP)
    pred_ctr_x = dx * widths + ctr_x
    pred_ctr_y = dy * heights + ctr_y
    pred_w = jnp.exp(dw) * widths
    pred_h = jnp.exp(dh) * heights
    x1 = jnp.clip(pred_ctr_x - 0.5 * pred_w, 0.0, _IMG_W)
    y1 = jnp.clip(pred_ctr_y - 0.5 * pred_h, 0.0, _IMG_H)
    x2 = jnp.clip(pred_ctr_x + 0.5 * pred_w, 0.0, _IMG_W)
    y2 = jnp.clip(pred_ctr_y + 0.5 * pred_h, 0.0, _IMG_H)
    box_ref[0] = jnp.concatenate([x1, y1, x2, y2], axis=-1)


def _prep(reg, logits, anchors):
    return pl.pallas_call(
        _prep_body,
        grid=(_B,),
        in_specs=[
            pl.BlockSpec((1, _A, 4), lambda b: (b, 0, 0)),
            pl.BlockSpec((1, _A, _C), lambda b: (b, 0, 0)),
            pl.BlockSpec((1, _A, 4), lambda b: (b, 0, 0)),
        ],
        out_specs=[
            pl.BlockSpec((1, _A, _C), lambda b: (b, 0, 0)),
            pl.BlockSpec((1, _A, 4), lambda b: (b, 0, 0)),
        ],
        out_shape=[
            jax.ShapeDtypeStruct((_B, _A, _C), jnp.int32),
            jax.ShapeDtypeStruct((_B, _A, 4), jnp.float32),
        ],
    )(reg, logits, anchors)


def _cutoff_body(keys_ref, out_ref):
    keys = keys_ref[0]
    lo = jnp.full((_NCLS, 1), -2, jnp.int32)
    hi = jnp.full((_NCLS, 1), _ONE_KEY, jnp.int32)

    def it(_, c):
        lo, hi = c
        mid = lo + (hi - lo) // 2
        cnt = jnp.sum((keys > mid).astype(jnp.int32), axis=1, keepdims=True)
        pred = cnt < _TOPK
        return (jnp.where(pred, lo, mid), jnp.where(pred, mid, hi))

    lo, hi = jax.lax.fori_loop(0, 31, it, (lo, hi))
    cgt = jnp.sum((keys > hi).astype(jnp.int32), axis=1, keepdims=True)
    out_ref[0] = jnp.concatenate([hi, _TOPK - cgt], axis=1)


def _cutoff(keys_rows):
    return pl.pallas_call(
        _cutoff_body,
        grid=(_B,),
        in_specs=[pl.BlockSpec((1, _NCLS, _APAD), lambda b: (b, 0, 0))],
        out_specs=[pl.BlockSpec((1, _NCLS, 2), lambda b: (b, 0, 0))],
        out_shape=[jax.ShapeDtypeStruct((_B, _NCLS, 2), jnp.int32)],
    )(keys_rows)


def _compact_body(keys_hbm, v_hbm, need_hbm, okey_hbm, oidx_hbm,
                  row_v, okey_v, oidx_v, v_v, need_v):
    wid = jax.lax.axis_index("s") * 2 + jax.lax.axis_index("c")
    pltpu.sync_copy(v_hbm, v_v)
    pltpu.sync_copy(need_hbm, need_v)

    def process(r):
        pltpu.sync_copy(keys_hbm.at[r], row_v)
        rvec = jnp.zeros((16,), jnp.int32) + r
        vv = plsc.load_gather(v_v, [rvec])
        nv = plsc.load_gather(need_v, [rvec])

        def chunk(t, carry):
            ptr, tie = carry
            k = row_v[pl.ds(t * 16, 16)]
            gt = k > vv
            eq = k == vv
            pre = plsc.cumsum(eq.astype(jnp.int32))
            take = jnp.logical_and(eq, (pre + tie) <= nv)
            sel = jnp.logical_or(gt, take)
            pos = ptr + plsc.cumsum(sel.astype(jnp.int32)) - 1
            idxv = jax.lax.iota(jnp.int32, 16) + t * 16
            plsc.store_scatter(okey_v, [pos], k, mask=sel)
            plsc.store_scatter(oidx_v, [pos], idxv, mask=sel)
            return (ptr + plsc.all_reduce_population_count(sel),
                    tie + plsc.all_reduce_population_count(take))

        zero = jnp.zeros((16,), jnp.int32)
        jax.lax.fori_loop(0, _CHUNKS, chunk, (zero, zero))
        pltpu.sync_copy(okey_v, okey_hbm.at[r])
        pltpu.sync_copy(oidx_v, oidx_hbm.at[r])

    for t in range(11):
        process(wid + _NW * t)

    @pl.when(wid < _NROWSEL - 11 * _NW)
    def _():
        process(wid + 11 * _NW)


@functools.partial(
    pl.kernel,
    out_type=(jax.ShapeDtypeStruct((_NROWSEL, _TOPK), jnp.int32),
              jax.ShapeDtypeStruct((_NROWSEL, _TOPK), jnp.int32)),
    mesh=plsc.VectorSubcoreMesh(core_axis_name="c", subcore_axis_name="s"),
    compiler_params=pltpu.CompilerParams(needs_layout_passes=False),
    scratch_types=[
        pltpu.VMEM((_APAD,), jnp.int32),
        pltpu.VMEM((_TOPK,), jnp.int32),
        pltpu.VMEM((_TOPK,), jnp.int32),
        pltpu.VMEM((_NROWSEL,), jnp.int32),
        pltpu.VMEM((_NROWSEL,), jnp.int32),
    ],
)
def _compact(keys_hbm, v_hbm, need_hbm, okey_hbm, oidx_hbm,
             row_v, okey_v, oidx_v, v_v, need_v):
    _compact_body(keys_hbm, v_hbm, need_hbm, okey_hbm, oidx_hbm,
                  row_v, okey_v, oidx_v, v_v, need_v)


def _sort_body(key_ref, idx_ref, oidx_ref, oscore_ref):
    k = key_ref[0]
    ix = idx_ref[0]
    lane = jax.lax.broadcasted_iota(jnp.int32, (_NCLS, 512), 1)
    size = 2
    while size <= 512:
        j = size // 2
        while j >= 1:
            bit = (lane & j) != 0
            up = (lane & size) == 0
            pk = jnp.where(bit, jnp.roll(k, j, 1), jnp.roll(k, -j, 1))
            pix = jnp.where(bit, jnp.roll(ix, j, 1), jnp.roll(ix, -j, 1))
            first = (k > pk) | ((k == pk) & (ix < pix))
            keep = first ^ (bit == up)
            k = jnp.where(keep, k, pk)
            ix = jnp.where(keep, ix, pix)
            j //= 2
        size *= 2
    oidx_ref[0] = ix
    oscore_ref[0] = jnp.where(
        k < 0, _NEG_INF, jax.lax.bitcast_convert_type(k, jnp.float32))


def _sort400(keys3, idx3):
    return pl.pallas_call(
        _sort_body,
        grid=(_B,),
        in_specs=[pl.BlockSpec((1, _NCLS, 512), lambda b: (b, 0, 0))] * 2,
        out_specs=[pl.BlockSpec((1, _NCLS, 512), lambda b: (b, 0, 0))] * 2,
        out_shape=[
            jax.ShapeDtypeStruct((_B, _NCLS, 512), jnp.int32),
            jax.ShapeDtypeStruct((_B, _NCLS, 512), jnp.float32),
        ],
    )(keys3, idx3)


_STRIPE = _NPAD // _NW


def _gather_body(boxes_hbm, gidx_hbm, planes_hbm, bbuf, idxbuf, obuf):
    wid = jax.lax.axis_index("s") * 2 + jax.lax.axis_index("c")
    base = wid * _STRIPE
    for b in range(_B):
        pltpu.sync_copy(boxes_hbm.at[b], bbuf)
        pltpu.sync_copy(gidx_hbm.at[b, pl.ds(base, _STRIPE)], idxbuf)

        def chunk(t, _):
            iv = idxbuf[pl.ds(t * 16, 16)] * 4
            for p in range(4):
                vals = plsc.load_gather(bbuf, [iv + p])
                obuf[pl.ds(p * _STRIPE + t * 16, 16)] = vals
            return 0

        jax.lax.fori_loop(0, _STRIPE // 16, chunk, 0)
        for p in range(4):
            pltpu.sync_copy(obuf.at[pl.ds(p * _STRIPE, _STRIPE)],
                            planes_hbm.at[b, p, pl.ds(base, _STRIPE)])


@functools.partial(
    pl.kernel,
    out_type=jax.ShapeDtypeStruct((_B, 4, _NPAD), jnp.float32),
    mesh=plsc.VectorSubcoreMesh(core_axis_name="c", subcore_axis_name="s"),
    compiler_params=pltpu.CompilerParams(needs_layout_passes=False),
    scratch_types=[
        pltpu.VMEM((_A * 4,), jnp.float32),
        pltpu.VMEM((_STRIPE,), jnp.int32),
        pltpu.VMEM((4 * _STRIPE,), jnp.float32),
    ],
)
def _gather_boxes(boxes_hbm, gidx_hbm, planes_hbm, bbuf, idxbuf, obuf):
    _gather_body(boxes_hbm, gidx_hbm, planes_hbm, bbuf, idxbuf, obuf)


def _nms_body(score_ref, x1_ref, y1_ref, x2_ref, y2_ref,
              obox_ref, oscore_ref, olabel_ref,
              s_ref, xo1_ref, yo1_ref, xo2_ref, yo2_ref, area_ref, iota_ref):
    iota = (jax.lax.broadcasted_iota(jnp.int32, (_NROWS, 128), 0) * 128
            + jax.lax.broadcasted_iota(jnp.int32, (_NROWS, 128), 1))
    iota_ref[...] = iota
    s_ref[...] = score_ref[0]
    x1 = x1_ref[0]
    y1 = y1_ref[0]
    x2 = x2_ref[0]
    y2 = y2_ref[0]
    max_coord = jnp.maximum(jnp.maximum(jnp.max(x1), jnp.max(y1)),
                            jnp.maximum(jnp.max(x2), jnp.max(y2)))
    labels = (iota // _TOPK + 1)
    off = labels.astype(jnp.float32) * (max_coord + 1.0)
    xo1 = x1 + off
    yo1 = y1 + off
    xo2 = x2 + off
    yo2 = y2 + off
    xo1_ref[...] = xo1
    yo1_ref[...] = yo1
    xo2_ref[...] = xo2
    yo2_ref[...] = yo2
    area_ref[...] = (xo2 - xo1) * (yo2 - yo1)

    def body(i, _):
        s = s_ref[...]
        m = jnp.max(s)
        best = jnp.min(jnp.where(s == m, iota_ref[...], jnp.int32(2**30)))
        r = best // 128
        c = jnp.remainder(best, 128)
        rs = pl.ds(r, 1)
        eq = jax.lax.broadcasted_iota(jnp.int32, (1, 128), 1) == c

        def fetch(row):
            return jnp.sum(jnp.where(eq, row, 0.0), axis=1, keepdims=True)

        bx1 = fetch(xo1_ref[rs, :])
        by1 = fetch(yo1_ref[rs, :])
        bx2 = fetch(xo2_ref[rs, :])
        by2 = fetch(yo2_ref[rs, :])
        area1 = (bx2 - bx1) * (by2 - by1)
        ltx = jnp.maximum(bx1, xo1_ref[...])
        lty = jnp.maximum(by1, yo1_ref[...])
        rbx = jnp.minimum(bx2, xo2_ref[...])
        rby = jnp.minimum(by2, yo2_ref[...])
        w = jnp.maximum(rbx - ltx, 0.0)
        h = jnp.maximum(rby - lty, 0.0)
        inter = w * h
        iou = inter / (area1 + area_ref[...] - inter + 1e-12)
        s = jnp.where(iou > _NMS_THRESH, _NEG_INF, s)
        s_ref[...] = s
        s_ref[rs, :] = jnp.where(eq, _NEG_INF, s_ref[rs, :])
        ob = jnp.concatenate([fetch(x1_ref[0, rs, :]), fetch(y1_ref[0, rs, :]),
                              fetch(x2_ref[0, rs, :]), fetch(y2_ref[0, rs, :])],
                             axis=-1)
        obox_ref[0, pl.ds(i, 1), :] = ob
        oscore_ref[0, pl.ds(i, 1), :] = fetch(score_ref[0, rs, :])
        olabel_ref[0, pl.ds(i, 1), :] = jnp.reshape(best // _TOPK + 1, (1, 1))
        return 0

    jax.lax.fori_loop(0, _DETS, body, 0)


def _nms(scores_pad, x1, y1, x2, y2):
    return pl.pallas_call(
        _nms_body,
        grid=(_B,),
        in_specs=[pl.BlockSpec((1, _NROWS, 128), lambda b: (b, 0, 0))] * 5,
        out_specs=[
            pl.BlockSpec((1, _DETS, 4), lambda b: (b, 0, 0)),
            pl.BlockSpec((1, _DETS, 1), lambda b: (b, 0, 0)),
            pl.BlockSpec((1, _DETS, 1), lambda b: (b, 0, 0)),
        ],
        out_shape=[
            jax.ShapeDtypeStruct((_B, _DETS, 4), jnp.float32),
            jax.ShapeDtypeStruct((_B, _DETS, 1), jnp.float32),
            jax.ShapeDtypeStruct((_B, _DETS, 1), jnp.int32),
        ],
        scratch_shapes=[pltpu.VMEM((_NROWS, 128), jnp.float32)] * 6
        + [pltpu.VMEM((_NROWS, 128), jnp.int32)],
    )(scores_pad, x1, y1, x2, y2)


def kernel(bbox_regression, cls_logits, anchors):
    keys, boxes = _prep(bbox_regression, cls_logits, anchors)

    keys_p = jnp.pad(keys, ((0, 0), (0, _APAD - _A), (0, 0)),
                     constant_values=-1)
    keys_rows4 = jnp.swapaxes(keys_p, 1, 2)[:, 1:, :]
    cut = _cutoff(keys_rows4)[0]

    keys_rows = keys_rows4.reshape(_NROWSEL, _APAD)
    v_arr = cut[:, :, 0].reshape(_NROWSEL)
    need_arr = cut[:, :, 1].reshape(_NROWSEL)
    ckey, cidx = _compact(keys_rows, v_arr, need_arr)

    ckey3 = jnp.pad(ckey.reshape(_B, _NCLS, _TOPK),
                    ((0, 0), (0, 0), (0, 112)), constant_values=-2)
    pad_idx = jnp.broadcast_to(jnp.arange(_APAD, _APAD + 112, dtype=jnp.int32),
                               (_B, _NCLS, 112))
    cidx3 = jnp.concatenate(
        [cidx.reshape(_B, _NCLS, _TOPK), pad_idx], axis=2)
    sidx, sscore = _sort400(ckey3, cidx3)

    gidx = jnp.pad(sidx[:, :, :_TOPK].reshape(_B, _NCAND),
                   ((0, 0), (0, _NPAD - _NCAND)))
    planes = _gather_boxes(boxes.reshape(_B, _A * 4), gidx)

    scores_pad = jnp.pad(sscore[:, :, :_TOPK].reshape(_B, _NCAND),
                         ((0, 0), (0, _NPAD - _NCAND)),
                         constant_values=_NEG_INF).reshape(_B, _NROWS, 128)
    planes = planes.reshape(_B, 4, _NROWS, 128)
    ob, osc, olb = _nms(scores_pad, planes[:, 0], planes[:, 1],
                        planes[:, 2], planes[:, 3])
    return ob, osc[..., 0], olb[..., 0]
